# baseline (speedup 1.0000x reference)
import functools

import jax
import jax.numpy as jnp
from jax import lax
from jax.experimental import pallas as pl
from jax.experimental.pallas import tpu as pltpu
from jax.experimental.pallas import tpu_sc as plsc

B = 16384
NW = 32
BPW = B // NW
L = 16
CHUNKS = BPW // L
OUT_D = 43
PRE_D = 24

COL_SPEC = (
    ("div", 0, 78.0), ("div", 1, 20.0),
    ("id", 2), ("id", 3), ("id", 4), ("id", 5), ("id", 6), ("id", 7), ("id", 8),
    ("lut", 9),
    ("ratio", 10, 11),
    ("id", 12),
    ("lut", 13),
    ("ratio", 14, 15),
    ("id", 16), ("id", 17), ("id", 18),
    ("lut", 19),
    ("id", 22),
)

_MESH = plsc.VectorSubcoreMesh(core_axis_name="c", subcore_axis_name="s")
_PARAMS = pltpu.CompilerParams(
    needs_layout_passes=False, use_tc_tiling_on_sc=False)


def _splat(val, dtype=jnp.float32):
    return jnp.broadcast_to(jnp.asarray(val, dtype), (L,))


def _const_vec(k):
    return jnp.full((L,), k, jnp.int32)


def _rsqrt16(v):
    i = plsc.bitcast(v, jnp.int32)
    i = jnp.int32(0x5F3759DF) - lax.shift_right_logical(i, 1)
    y = plsc.bitcast(i, jnp.float32)
    for _ in range(3):
        y = y * (jnp.float32(1.5) - jnp.float32(0.5) * v * y * y)
    return y


@functools.partial(
    pl.kernel,
    mesh=_MESH,
    out_type=[
        jax.ShapeDtypeStruct((B * PRE_D,), jnp.float32),
        jax.ShapeDtypeStruct((NW * 64,), jnp.float32),
    ],
    scratch_types=[
        pltpu.VMEM((BPW, 27), jnp.int32),
        pltpu.VMEM((8192,), jnp.float32),
        pltpu.VMEM((BPW * PRE_D,), jnp.float32),
        pltpu.VMEM((64,), jnp.float32),
    ],
    compiler_params=_PARAMS,
)
def _stage_a(bl_hbm, lut_hbm, pre_hbm, parts_hbm, blv, lutv, prev, statsv):
    wid = lax.axis_index("s") * 2 + lax.axis_index("c")
    pltpu.sync_copy(bl_hbm.at[pl.ds(wid * BPW, BPW)], blv)
    pltpu.sync_copy(lut_hbm, lutv)

    iota16 = lax.iota(jnp.int32, L)
    lane0 = iota16 == 0
    zero = _splat(0.0)

    def gather_col(rows, col):
        return plsc.load_gather(blv, [rows, _const_vec(col)])

    UNROLL = 4

    for k, spec in enumerate(COL_SPEC):
        kind = spec[0]

        def chunk(c, carry, kind=kind, spec=spec, k=k):
            s, q = carry
            for j in range(UNROLL):
                rows = (c * UNROLL + j) * L + iota16
                if kind == "div":
                    y = gather_col(rows, spec[1]).astype(jnp.float32) / _splat(spec[2])
                elif kind == "id":
                    y = gather_col(rows, spec[1]).astype(jnp.float32)
                elif kind == "lut":
                    idx = gather_col(rows, spec[1])
                    idx = jnp.clip(idx, 0, 8191)
                    y = plsc.load_gather(lutv, [idx])
                else:
                    a = gather_col(rows, spec[1]).astype(jnp.float32)
                    b = gather_col(rows, spec[2]).astype(jnp.float32)
                    y = a / jnp.maximum(b, _splat(1.0))
                plsc.store_scatter(prev, [rows * PRE_D + k], y)
                s = s + y
                q = q + y * y
            return (s, q)

        s, q = lax.fori_loop(0, CHUNKS // UNROLL, chunk, (zero, zero))
        plsc.store_scatter(
            statsv, [_const_vec(k)],
            jnp.broadcast_to(jnp.sum(s), (L,)), mask=lane0)
        plsc.store_scatter(
            statsv, [_const_vec(32 + k)],
            jnp.broadcast_to(jnp.sum(q), (L,)), mask=lane0)

    def pass_chunk(c, carry):
        for j in range(2):
            rows = (c * 2 + j) * L + iota16
            for dst, src, hi in ((19, 21, 6), (20, 23, 10), (21, 24, 50), (22, 25, None)):
                v = gather_col(rows, src)
                if hi is not None:
                    v = jnp.clip(v, 0, hi)
                plsc.store_scatter(prev, [rows * PRE_D + dst], v.astype(jnp.float32))
        return carry

    lax.fori_loop(0, CHUNKS // 2, pass_chunk, 0)

    pltpu.sync_copy(prev, pre_hbm.at[pl.ds(wid * (BPW * PRE_D), BPW * PRE_D)])
    pltpu.sync_copy(statsv, parts_hbm.at[pl.ds(wid * 64, 64)])


@functools.partial(
    pl.kernel,
    mesh=_MESH,
    out_type=jax.ShapeDtypeStruct((B, OUT_D), jnp.float32),
    scratch_types=[
        pltpu.VMEM((BPW * PRE_D,), jnp.float32),
        pltpu.VMEM((NW * 64,), jnp.float32),
        pltpu.VMEM((32,), jnp.float32),
        pltpu.VMEM((32,), jnp.float32),
        pltpu.VMEM((24,), jnp.float32),
        pltpu.VMEM((48,), jnp.float32),
        pltpu.VMEM((208,), jnp.float32),
        pltpu.VMEM((BPW, OUT_D), jnp.float32),
    ],
    compiler_params=_PARAMS,
)
def _stage_b(pre_hbm, parts_hbm, bw_hbm, bb_hbm, ht_hbm, dt_hbm, lt_hbm,
             out_hbm, prev, partsv, bwv, bbv, htv, dtv, ltv, outv):
    wid = lax.axis_index("s") * 2 + lax.axis_index("c")
    pltpu.sync_copy(pre_hbm.at[pl.ds(wid * (BPW * PRE_D), BPW * PRE_D)], prev)
    pltpu.sync_copy(parts_hbm, partsv)
    pltpu.sync_copy(bw_hbm, bwv)
    pltpu.sync_copy(bb_hbm, bbv)
    pltpu.sync_copy(ht_hbm, htv)
    pltpu.sync_copy(dt_hbm, dtv)
    pltpu.sync_copy(lt_hbm, ltv)

    iota16 = lax.iota(jnp.int32, L)
    inv_n = _splat(1.0 / B)
    zero = _splat(0.0)

    tot = [zero, zero, zero, zero]
    for w in range(NW):
        for g in range(4):
            tot[g] = tot[g] + partsv[pl.ds(w * 64 + g * 16, L)]
    scale_g, shift_g = [], []
    for g in range(2):
        mean = tot[g] * inv_n
        ex2 = tot[2 + g] * inv_n
        var = jnp.maximum(ex2 - mean * mean, _splat(0.0)) + _splat(1e-5)
        inv = _rsqrt16(var)
        scale = bwv[pl.ds(g * 16, L)] * inv
        shift = bbv[pl.ds(g * 16, L)] - mean * scale
        scale_g.append(scale)
        shift_g.append(shift)

    for k in range(19):
        sel = iota16 == (k % 16)
        scale = jnp.broadcast_to(
            jnp.sum(jnp.where(sel, scale_g[k // 16], zero)), (L,))
        shift = jnp.broadcast_to(
            jnp.sum(jnp.where(sel, shift_g[k // 16], zero)), (L,))

        def norm_chunk(c, carry, k=k, scale=scale, shift=shift):
            for j in range(4):
                rows = (c * 4 + j) * L + iota16
                x = plsc.load_gather(prev, [rows * PRE_D + k])
                plsc.store_scatter(outv, [rows, _const_vec(k)], x * scale + shift)
            return carry

        lax.fori_loop(0, CHUNKS // 4, norm_chunk, 0)

    def tail_chunk(c, carry):
        for j in range(2):
            rows = (c * 2 + j) * L + iota16
            rows_pre = rows * PRE_D
            h = plsc.load_gather(prev, [rows_pre + 19]).astype(jnp.int32)
            for dd in range(3):
                e = plsc.load_gather(htv, [h * 3 + dd])
                plsc.store_scatter(outv, [rows, _const_vec(19 + dd)], e)
            dg = plsc.load_gather(prev, [rows_pre + 20]).astype(jnp.int32)
            for dd in range(4):
                e = plsc.load_gather(dtv, [dg * 4 + dd])
                plsc.store_scatter(outv, [rows, _const_vec(22 + dd)], e)
            lv = plsc.load_gather(prev, [rows_pre + 21]).astype(jnp.int32)
            for dd in range(4):
                e = plsc.load_gather(ltv, [lv * 4 + dd])
                plsc.store_scatter(outv, [rows, _const_vec(26 + dd)], e)
            m = plsc.load_gather(prev, [rows_pre + 22]).astype(jnp.int32)
            for kk in range(13):
                bit = (lax.shift_right_logical(m, kk) & 1).astype(jnp.float32)
                plsc.store_scatter(outv, [rows, _const_vec(30 + kk)], bit)
        return carry

    lax.fori_loop(0, CHUNKS // 2, tail_chunk, 0)

    pltpu.sync_copy(outv, out_hbm.at[pl.ds(wid * BPW, BPW)])


def kernel(bl, bn_weight, bn_bias, hunger_table, dungeon_table, level_table):
    bl = bl.astype(jnp.int32)
    lut = jnp.log1p(jnp.arange(8192, dtype=jnp.float32))
    htab = jnp.pad(jnp.ravel(hunger_table.astype(jnp.float32)), (0, 3))
    dtab = jnp.pad(jnp.ravel(dungeon_table.astype(jnp.float32)), (0, 4))
    ltab = jnp.pad(jnp.ravel(level_table.astype(jnp.float32)), (0, 4))
    bw = jnp.pad(bn_weight.astype(jnp.float32), (0, 13))
    bb = jnp.pad(bn_bias.astype(jnp.float32), (0, 13))
    pre, parts = _stage_a(bl, lut)
    return _stage_b(pre, parts, bw, bb, htab, dtab, ltab)


# speedup vs baseline: 1.2067x; 1.2067x over previous
import functools

import jax
import jax.numpy as jnp
from jax import lax
from jax.experimental import pallas as pl
from jax.experimental.pallas import tpu as pltpu
from jax.experimental.pallas import tpu_sc as plsc

B = 16384
NW = 32
BPW = B // NW
L = 16
CHUNKS = BPW // L
OUT_D = 43

COL_SPEC = (
    ("div", 0, 78.0), ("div", 1, 20.0),
    ("id", 2), ("id", 3), ("id", 4), ("id", 5), ("id", 6), ("id", 7), ("id", 8),
    ("lut", 9),
    ("ratio", 10, 11),
    ("id", 12),
    ("lut", 13),
    ("ratio", 14, 15),
    ("id", 16), ("id", 17), ("id", 18),
    ("lut", 19),
    ("id", 22),
)

_MESH = plsc.VectorSubcoreMesh(core_axis_name="c", subcore_axis_name="s")
_PARAMS = pltpu.CompilerParams(needs_layout_passes=False)


def _splat(val, dtype=jnp.float32):
    return jnp.broadcast_to(jnp.asarray(val, dtype), (L,))


def _const_vec(k):
    return jnp.full((L,), k, jnp.int32)


@functools.partial(
    pl.kernel,
    mesh=_MESH,
    out_type=[
        jax.ShapeDtypeStruct((B, OUT_D), jnp.float32),
        jax.ShapeDtypeStruct((NW * 64,), jnp.float32),
    ],
    scratch_types=[
        pltpu.VMEM((BPW, 27), jnp.int32),
        pltpu.VMEM((8192,), jnp.float32),
        pltpu.VMEM((24,), jnp.float32),
        pltpu.VMEM((48,), jnp.float32),
        pltpu.VMEM((208,), jnp.float32),
        pltpu.VMEM((BPW // 2, OUT_D), jnp.float32),
        pltpu.VMEM((64,), jnp.float32),
    ],
    compiler_params=_PARAMS,
)
def _sc_stage(bl_hbm, lut_hbm, ht_hbm, dt_hbm, lt_hbm,
              out_hbm, parts_hbm, blv, lutv, htv, dtv, ltv, outv, statsv):
    wid = lax.axis_index("s") * 2 + lax.axis_index("c")
    pltpu.sync_copy(bl_hbm.at[pl.ds(wid * BPW, BPW)], blv)
    pltpu.sync_copy(lut_hbm, lutv)
    pltpu.sync_copy(ht_hbm, htv)
    pltpu.sync_copy(dt_hbm, dtv)
    pltpu.sync_copy(lt_hbm, ltv)

    iota16 = lax.iota(jnp.int32, L)
    lane0 = iota16 == 0
    zero = _splat(0.0)

    def gather_col(rows, col):
        return plsc.load_gather(blv, [rows, _const_vec(col)])

    UNROLL = 4
    HALF = BPW // 2
    HCHUNKS = HALF // L

    accs = {k: (zero, zero) for k in range(len(COL_SPEC))}
    for half in range(2):
        row0 = half * HALF

        for k, spec in enumerate(COL_SPEC):
            kind = spec[0]

            def chunk(c, carry, kind=kind, spec=spec, k=k, row0=row0):
                s, q = carry
                for j in range(UNROLL):
                    local = (c * UNROLL + j) * L + iota16
                    rows = local + row0
                    if kind == "div":
                        y = gather_col(rows, spec[1]).astype(jnp.float32) / _splat(spec[2])
                    elif kind == "id":
                        y = gather_col(rows, spec[1]).astype(jnp.float32)
                    elif kind == "lut":
                        idx = gather_col(rows, spec[1])
                        idx = jnp.clip(idx, 0, 8191)
                        y = plsc.load_gather(lutv, [idx])
                    else:
                        a = gather_col(rows, spec[1]).astype(jnp.float32)
                        b = gather_col(rows, spec[2]).astype(jnp.float32)
                        y = a / jnp.maximum(b, _splat(1.0))
                    plsc.store_scatter(outv, [local, _const_vec(k)], y)
                    s = s + y
                    q = q + y * y
                return (s, q)

            accs[k] = lax.fori_loop(0, HCHUNKS // UNROLL, chunk, accs[k])

        def tail_chunk(c, carry, row0=row0):
            for j in range(2):
                local = (c * 2 + j) * L + iota16
                rows = local + row0
                h = jnp.clip(gather_col(rows, 21), 0, 6)
                for dd in range(3):
                    e = plsc.load_gather(htv, [h * 3 + dd])
                    plsc.store_scatter(outv, [local, _const_vec(19 + dd)], e)
                dg = jnp.clip(gather_col(rows, 23), 0, 10)
                for dd in range(4):
                    e = plsc.load_gather(dtv, [dg * 4 + dd])
                    plsc.store_scatter(outv, [local, _const_vec(22 + dd)], e)
                lv = jnp.clip(gather_col(rows, 24), 0, 50)
                for dd in range(4):
                    e = plsc.load_gather(ltv, [lv * 4 + dd])
                    plsc.store_scatter(outv, [local, _const_vec(26 + dd)], e)
                m = gather_col(rows, 25)
                for kk in range(13):
                    bit = (lax.shift_right_logical(m, kk) & 1).astype(jnp.float32)
                    plsc.store_scatter(outv, [local, _const_vec(30 + kk)], bit)
            return carry

        lax.fori_loop(0, HCHUNKS // 2, tail_chunk, 0)

        pltpu.sync_copy(outv, out_hbm.at[pl.ds(wid * BPW + row0, HALF)])

    for k in range(len(COL_SPEC)):
        s, q = accs[k]
        plsc.store_scatter(
            statsv, [_const_vec(k)],
            jnp.broadcast_to(jnp.sum(s), (L,)), mask=lane0)
        plsc.store_scatter(
            statsv, [_const_vec(32 + k)],
            jnp.broadcast_to(jnp.sum(q), (L,)), mask=lane0)

    pltpu.sync_copy(statsv, parts_hbm.at[pl.ds(wid * 64, 64)])


def _tc_norm_body(raw_ref, parts_ref, bw_ref, bb_ref, out_ref):
    parts = parts_ref[...]
    tot = jnp.sum(parts, axis=0, keepdims=True)
    tot64 = lax.slice(tot, (0, 0), (1, 64)) + lax.slice(tot, (0, 64), (1, 128))
    sq64 = jnp.roll(tot64, -32, axis=1)
    inv_n = jnp.float32(1.0 / B)
    mean = tot64 * inv_n
    ex2 = sq64 * inv_n
    var = jnp.maximum(ex2 - mean * mean, 0.0) + jnp.float32(1e-5)
    inv = lax.rsqrt(var)
    lane = lax.broadcasted_iota(jnp.int32, (1, 64), 1)
    is_cont = lane < 19
    scale = jnp.where(is_cont, bw_ref[...] * inv, 1.0)
    shift = jnp.where(is_cont, bb_ref[...] - mean * scale, 0.0)
    scale43 = lax.slice(scale, (0, 0), (1, OUT_D))
    shift43 = lax.slice(shift, (0, 0), (1, OUT_D))
    out_ref[...] = raw_ref[...] * scale43 + shift43


_tc_norm = pl.pallas_call(
    _tc_norm_body,
    out_shape=jax.ShapeDtypeStruct((B, OUT_D), jnp.float32),
)


def kernel(bl, bn_weight, bn_bias, hunger_table, dungeon_table, level_table):
    bl = bl.astype(jnp.int32)
    lut = jnp.log1p(jnp.arange(8192, dtype=jnp.float32))
    htab = jnp.pad(jnp.ravel(hunger_table.astype(jnp.float32)), (0, 3))
    dtab = jnp.pad(jnp.ravel(dungeon_table.astype(jnp.float32)), (0, 4))
    ltab = jnp.pad(jnp.ravel(level_table.astype(jnp.float32)), (0, 4))
    bw = jnp.pad(bn_weight.astype(jnp.float32), (0, 45)).reshape(1, 64)
    bb = jnp.pad(bn_bias.astype(jnp.float32), (0, 45)).reshape(1, 64)
    raw, parts = _sc_stage(bl, lut, htab, dtab, ltab)
    return _tc_norm(raw, parts.reshape(16, 128), bw, bb)
